# baseline (speedup 1.0000x reference)
import functools

import jax
import jax.numpy as jnp
from jax import lax
from jax.experimental import pallas as pl
from jax.experimental.pallas import tpu as pltpu
from jax.experimental.pallas import tpu_sc as plsc

_N = 10000
_E = 320000
_D = 128
_DE = 16
_BN_EPS = 1e-5

_NC = 2
_NS = 16
_L = 16

_NW = _NC * _NS
_K = 128
_NCHUNKS = _E // _K
_BASE_SLOTS = _NCHUNKS // _NW
_EXTRA = _NCHUNKS - _BASE_SLOTS * _NW
_NSLOT = _BASE_SLOTS + 1
_NP = 3584
_RNG = 3456
_NPASS = 3
_NO = 10496
_RPS = _NP // _NS
_ZB = 32

_PACK = 8
_EP = _E // _PACK
_BK = 2000


def _enc_body(a_ref, k_ref, b_ref, o_ref):
    a = a_ref[...].astype(jnp.bfloat16)
    o_ref[...] = (
        jnp.dot(a, k_ref[...], preferred_element_type=jnp.float32) + b_ref[...]
    )


def _edge_encode(attr_packed, k_big, bias_tiled):
    return pl.pallas_call(
        _enc_body,
        grid=(_EP // _BK,),
        in_specs=[
            pl.BlockSpec((_BK, _D), lambda i: (i, 0)),
            pl.BlockSpec((_D, _PACK * _D), lambda i: (0, 0)),
            pl.BlockSpec((1, _PACK * _D), lambda i: (0, 0)),
        ],
        out_specs=pl.BlockSpec((_BK, _PACK * _D), lambda i: (i, 0)),
        out_shape=jax.ShapeDtypeStruct((_EP, _PACK * _D), jnp.float32),
    )(attr_packed, k_big, bias_tiled)


def _sc_body(x_hbm, eap_hbm, src_hbm, dst_hbm, out0_hbm, out1_hbm, msg_hbm,
             srcv0, srcv1, dstv0, dstv1, xg0, xg1, eav0, eav1, zb, agg,
             s_src0, s_src1, s_dst0, s_dst1, s_g0, s_g1, s_e0, s_e1,
             s_m0, s_m1, s_sc0, s_sc1):
    cid = lax.axis_index("c")
    sid = lax.axis_index("s")
    wid = sid * _NC + cid

    n_real = _BASE_SLOTS + jnp.where(wid < _EXTRA, 1, 0)
    start = wid * _BASE_SLOTS + jnp.minimum(wid, _EXTRA)

    def to_cid(s):
        return start + jnp.where(s < n_real, s, 0)

    sets = (
        (srcv0, dstv0, xg0, eav0, s_src0, s_dst0, s_g0, s_e0, s_m0, s_sc0),
        (srcv1, dstv1, xg1, eav1, s_src1, s_dst1, s_g1, s_e1, s_m1, s_sc1),
    )

    def start_src(b, s):
        srcv, s_src = sets[b][0], sets[b][4]
        pltpu.async_copy(src_hbm.at[pl.ds(to_cid(s) * _K, _K)], srcv, s_src)

    def wait_src(b):
        srcv, s_src = sets[b][0], sets[b][4]
        pltpu.make_async_copy(src_hbm.at[pl.ds(0, _K)], srcv, s_src).wait()

    def start_dst(b, s):
        dstv, s_dst = sets[b][1], sets[b][5]
        pltpu.async_copy(dst_hbm.at[pl.ds(to_cid(s) * _K, _K)], dstv, s_dst)

    def wait_dst(b):
        dstv, s_dst = sets[b][1], sets[b][5]
        pltpu.make_async_copy(dst_hbm.at[pl.ds(0, _K)], dstv, s_dst).wait()

    def start_data(b, s):
        srcv, xg, eav = sets[b][0], sets[b][2], sets[b][3]
        s_g, s_e = sets[b][6], sets[b][7]
        pltpu.async_copy(x_hbm.at[srcv], xg, s_g)
        pltpu.async_copy(eap_hbm.at[pl.ds(to_cid(s) * (_K // _PACK), _K // _PACK)],
                         eav, s_e)

    def wait_data(b):
        srcv, xg, eav = sets[b][0], sets[b][2], sets[b][3]
        s_g, s_e = sets[b][6], sets[b][7]
        pltpu.make_async_copy(x_hbm.at[srcv], xg, s_g).wait()
        pltpu.make_async_copy(eap_hbm.at[pl.ds(0, _K // _PACK)], eav, s_e).wait()

    def compute(b):
        xg, eav = sets[b][2], sets[b][3]

        @pl.loop(0, _K // _PACK)
        def _(pr):
            for a in range(_PACK):
                for cc in range(0, _D, _L):
                    v = (xg[pr * _PACK + a, pl.ds(cc, _L)]
                         + eav[pr, pl.ds(a * _D + cc, _L)])
                    xg[pr * _PACK + a, pl.ds(cc, _L)] = jnp.maximum(v, 0.0)

    def adjust_and_fire(b, s):
        dstv, xg = sets[b][1], sets[b][2]
        s_m, s_sc = sets[b][8], sets[b][9]

        wait_dst(b)

        @pl.when(s >= n_real)
        def _():
            for j in range(_K // _L):
                dstv[pl.ds(j * _L, _L)] = _N + j * _L + lax.iota(jnp.int32, _L)

        for j in range(_K // _L):
            d = dstv[pl.ds(j * _L, _L)]
            ok = d < _RNG
            pad = _RNG + j * _L + lax.iota(jnp.int32, _L)
            dstv[pl.ds(j * _L, _L)] = jnp.where(ok, d, pad)

        pltpu.async_copy(xg, msg_hbm.at[pl.ds(to_cid(s) * _K, _K)], s_m)
        pltpu.async_copy(xg, agg.at[dstv], s_sc, add=True)

    def wait_scatter(b):
        dstv, xg, s_sc = sets[b][1], sets[b][2], sets[b][9]
        pltpu.make_async_copy(xg, agg.at[dstv], s_sc).wait()

    def wait_msg(b):
        xg, s_m = sets[b][2], sets[b][8]
        pltpu.make_async_copy(xg, msg_hbm.at[pl.ds(0, _K)], s_m).wait()

    def start_light(b, s, p):
        dstv, xg = sets[b][1], sets[b][2]
        s_dst, s_g = sets[b][5], sets[b][6]
        pltpu.async_copy(dst_hbm.at[pl.ds(to_cid(s) * _K, _K)], dstv, s_dst)
        pltpu.async_copy(msg_hbm.at[pl.ds(to_cid(s) * _K, _K)], xg, s_g)

    def process_light(b, s, p):
        dstv, xg = sets[b][1], sets[b][2]
        s_dst, s_g = sets[b][5], sets[b][6]
        pltpu.make_async_copy(dst_hbm.at[pl.ds(0, _K)], dstv, s_dst).wait()
        pltpu.make_async_copy(msg_hbm.at[pl.ds(0, _K)], xg, s_g).wait()

        @pl.when(s >= n_real)
        def _():
            for j in range(_K // _L):
                dstv[pl.ds(j * _L, _L)] = _N + j * _L + lax.iota(jnp.int32, _L)

        for j in range(_K // _L):
            d = dstv[pl.ds(j * _L, _L)] - (p * _RNG)
            ok = (d >= 0) & (d < _RNG)
            pad = _RNG + j * _L + lax.iota(jnp.int32, _L)
            dstv[pl.ds(j * _L, _L)] = jnp.where(ok, d, pad)

        pltpu.sync_copy(xg, agg.at[dstv], add=True)

    @pl.loop(0, _ZB)
    def _(r):
        for cc in range(0, _D, _L):
            zb[r, pl.ds(cc, _L)] = jnp.zeros((_L,), jnp.float32)

    def copy_out(p):
        @pl.loop(0, _RPS // _ZB)
        def _(j):
            row = sid * _RPS + j * _ZB

            @pl.when(cid == 0)
            def _():
                pltpu.sync_copy(agg.at[pl.ds(row, _ZB)],
                                out0_hbm.at[pl.ds(p * _RNG + row, _ZB)])

            @pl.when(cid == 1)
            def _():
                pltpu.sync_copy(agg.at[pl.ds(row, _ZB)],
                                out1_hbm.at[pl.ds(p * _RNG + row, _ZB)])

    def zero_agg():
        @pl.loop(0, _RPS // _ZB)
        def _(j):
            pltpu.sync_copy(zb, agg.at[pl.ds(sid * _RPS + j * _ZB, _ZB)])

    zero_agg()
    plsc.subcore_barrier()

    start_src(0, 0)
    start_dst(0, 0)
    start_src(1, 1)
    wait_src(0)
    start_data(0, 0)

    @pl.loop(0, (_NSLOT - 1) // 2)
    def _(t):
        c0 = 2 * t
        c1 = c0 + 1
        wait_src(1)

        @pl.when(t > 0)
        def _():
            wait_msg(1)
            wait_scatter(1)

        start_data(1, c1)
        start_dst(1, c1)
        wait_data(0)
        start_src(0, c0 + 2)
        compute(0)
        adjust_and_fire(0, c0)
        wait_src(0)
        wait_data(1)
        compute(1)
        adjust_and_fire(1, c1)
        wait_msg(0)
        wait_scatter(0)
        start_data(0, c0 + 2)
        start_dst(0, c0 + 2)

        @pl.when(c1 + 2 < _NSLOT)
        def _():
            start_src(1, c1 + 2)

    wait_msg(1)
    wait_scatter(1)
    wait_data(0)
    compute(0)
    adjust_and_fire(0, _NSLOT - 1)
    wait_msg(0)
    wait_scatter(0)

    plsc.subcore_barrier()
    copy_out(0)
    plsc.subcore_barrier()

    @pl.loop(1, _NPASS)
    def _(p):
        zero_agg()
        plsc.subcore_barrier()

        start_light(0, 0, p)
        start_light(1, 1, p)

        @pl.loop(0, (_NSLOT - 1) // 2)
        def _(t):
            c0 = 2 * t
            c1 = c0 + 1
            process_light(0, c0, p)
            start_light(0, c0 + 2, p)
            process_light(1, c1, p)

            @pl.when(c1 + 2 < _NSLOT)
            def _():
                start_light(1, c1 + 2, p)

        process_light(0, _NSLOT - 1, p)

        plsc.subcore_barrier()
        copy_out(p)
        plsc.subcore_barrier()


def _sc_aggregate(x, eap, src, dst):
    mesh = plsc.VectorSubcoreMesh(
        core_axis_name="c", subcore_axis_name="s",
        num_cores=_NC, num_subcores=_NS,
    )
    f = pl.kernel(
        _sc_body,
        out_type=[
            jax.ShapeDtypeStruct((_NO, _D), jnp.float32),
            jax.ShapeDtypeStruct((_NO, _D), jnp.float32),
            jax.ShapeDtypeStruct((_E, _D), jnp.float32),
        ],
        mesh=mesh,
        scratch_types=[
            pltpu.VMEM((_K,), jnp.int32),
            pltpu.VMEM((_K,), jnp.int32),
            pltpu.VMEM((_K,), jnp.int32),
            pltpu.VMEM((_K,), jnp.int32),
            pltpu.VMEM((_K, _D), jnp.float32),
            pltpu.VMEM((_K, _D), jnp.float32),
            pltpu.VMEM((_K // _PACK, _PACK * _D), jnp.float32),
            pltpu.VMEM((_K // _PACK, _PACK * _D), jnp.float32),
            pltpu.VMEM((_ZB, _D), jnp.float32),
            pltpu.VMEM_SHARED((_NP, _D), jnp.float32),
            pltpu.SemaphoreType.DMA,
            pltpu.SemaphoreType.DMA,
            pltpu.SemaphoreType.DMA,
            pltpu.SemaphoreType.DMA,
            pltpu.SemaphoreType.DMA,
            pltpu.SemaphoreType.DMA,
            pltpu.SemaphoreType.DMA,
            pltpu.SemaphoreType.DMA,
            pltpu.SemaphoreType.DMA,
            pltpu.SemaphoreType.DMA,
            pltpu.SemaphoreType.DMA,
            pltpu.SemaphoreType.DMA,
        ],
    )
    return f(x, eap, src, dst)


def _mlp_body(eps_ref, x_ref, p0_ref, p1_ref, w1_ref, c1_ref, w2_ref, c2_ref,
              s3_ref, c3_ref, o_ref):
    a = (1.0 + eps_ref[0]) * x_ref[...] + p0_ref[...] + p1_ref[...]
    h = jnp.dot(a, w1_ref[...], preferred_element_type=jnp.float32) + c1_ref[...]
    h = jnp.maximum(h, 0.0)
    h = jnp.dot(h, w2_ref[...], preferred_element_type=jnp.float32) + c2_ref[...]
    h = jnp.maximum(h, 0.0)
    o_ref[...] = jnp.maximum(h * s3_ref[...] + c3_ref[...], 0.0)


def _mlp(x, p0, p1, eps_p, w1f, c1, w2f, c2, s3, c3):
    bn = 2000
    return pl.pallas_call(
        _mlp_body,
        grid=(_N // bn,),
        in_specs=[
            pl.BlockSpec(memory_space=pltpu.SMEM),
            pl.BlockSpec((bn, _D), lambda i: (i, 0)),
            pl.BlockSpec((bn, _D), lambda i: (i, 0)),
            pl.BlockSpec((bn, _D), lambda i: (i, 0)),
            pl.BlockSpec((_D, _D), lambda i: (0, 0)),
            pl.BlockSpec((1, _D), lambda i: (0, 0)),
            pl.BlockSpec((_D, _D), lambda i: (0, 0)),
            pl.BlockSpec((1, _D), lambda i: (0, 0)),
            pl.BlockSpec((1, _D), lambda i: (0, 0)),
            pl.BlockSpec((1, _D), lambda i: (0, 0)),
        ],
        out_specs=pl.BlockSpec((bn, _D), lambda i: (i, 0)),
        out_shape=jax.ShapeDtypeStruct((_N, _D), jnp.float32),
    )(eps_p.reshape(1), x, p0, p1, w1f, c1, w2f, c2, s3, c3)


def kernel(x, edge_index, edge_attr, W_enc, b_enc, W_lin, b_lin,
           W1, b1, g1, be1, W2, b2, g2, be2, g_bn, be_bn, eps_p):
    Wc = W_lin @ W_enc
    bc = W_lin @ b_enc + b_lin
    eye = jnp.eye(_PACK, dtype=jnp.float32)
    k_big = jnp.einsum("ab,jo->ajbo", eye, Wc.T)
    k_big = k_big.reshape(_PACK * _DE, _PACK * _D).astype(jnp.bfloat16)
    bias_tiled = jnp.tile(bc, _PACK).reshape(1, _PACK * _D)

    inv = 1.0 / jnp.sqrt(1.0 + _BN_EPS)
    w1f = W1.T * (inv * g1)[None, :]
    c1 = ((b1 * inv) * g1 + be1).reshape(1, _D)
    w2f = W2.T * (inv * g2)[None, :]
    c2 = ((b2 * inv) * g2 + be2).reshape(1, _D)
    s3 = (inv * g_bn).reshape(1, _D)
    c3 = be_bn.reshape(1, _D)

    attr_packed = edge_attr.reshape(_EP, _PACK * _DE)
    eap = _edge_encode(attr_packed, k_big, bias_tiled)

    src = edge_index[0]
    dst = edge_index[1]
    p0, p1, _msg = _sc_aggregate(x, eap, src, dst)

    return _mlp(x, p0, p1, eps_p, w1f, c1, w2f, c2, s3, c3)


# speedup vs baseline: 1.0197x; 1.0197x over previous
import functools

import jax
import jax.numpy as jnp
from jax import lax
from jax.experimental import pallas as pl
from jax.experimental.pallas import tpu as pltpu
from jax.experimental.pallas import tpu_sc as plsc

_N = 10000
_E = 320000
_D = 128
_DE = 16
_BN_EPS = 1e-5

_NC = 2
_NS = 16
_L = 16

_NW = _NC * _NS
_K = 128
_NCHUNKS = _E // _K
_BASE_SLOTS = _NCHUNKS // _NW
_EXTRA = _NCHUNKS - _BASE_SLOTS * _NW
_NSLOT = _BASE_SLOTS + 1
_NP = 3584
_RNG = 3456
_NPASS = 3
_NO = 10496
_RPS = _NP // _NS
_ZB = 32

_PACK = 8
_EP = _E // _PACK
_BK = 2000


def _enc_body(a_ref, k_ref, b_ref, o_ref):
    a = a_ref[...].astype(jnp.bfloat16)
    o_ref[...] = (
        jnp.dot(a, k_ref[...], preferred_element_type=jnp.float32) + b_ref[...]
    )


def _edge_encode(attr_packed, k_big, bias_tiled):
    return pl.pallas_call(
        _enc_body,
        grid=(_EP // _BK,),
        in_specs=[
            pl.BlockSpec((_BK, _D), lambda i: (i, 0)),
            pl.BlockSpec((_D, _PACK * _D), lambda i: (0, 0)),
            pl.BlockSpec((1, _PACK * _D), lambda i: (0, 0)),
        ],
        out_specs=pl.BlockSpec((_BK, _PACK * _D), lambda i: (i, 0)),
        out_shape=jax.ShapeDtypeStruct((_EP, _PACK * _D), jnp.float32),
    )(attr_packed, k_big, bias_tiled)


def _sc_body(x_hbm, eap_hbm, src_hbm, dst_hbm, out0_hbm, out1_hbm, msg_hbm,
             srcv0, srcv1, dstv0, dstv1, xg0, xg1, eav0, eav1, zb, agg,
             s_src0, s_src1, s_dst0, s_dst1, s_g0, s_g1, s_e0, s_e1,
             s_m0, s_m1):
    cid = lax.axis_index("c")
    sid = lax.axis_index("s")
    wid = sid * _NC + cid

    n_real = _BASE_SLOTS + jnp.where(wid < _EXTRA, 1, 0)
    start = wid * _BASE_SLOTS + jnp.minimum(wid, _EXTRA)

    def to_cid(s):
        return start + jnp.where(s < n_real, s, 0)

    sets = (
        (srcv0, dstv0, xg0, eav0, s_src0, s_dst0, s_g0, s_e0, s_m0),
        (srcv1, dstv1, xg1, eav1, s_src1, s_dst1, s_g1, s_e1, s_m1),
    )

    def start_src(b, s):
        srcv, s_src = sets[b][0], sets[b][4]
        pltpu.async_copy(src_hbm.at[pl.ds(to_cid(s) * _K, _K)], srcv, s_src)

    def wait_src(b):
        srcv, s_src = sets[b][0], sets[b][4]
        pltpu.make_async_copy(src_hbm.at[pl.ds(0, _K)], srcv, s_src).wait()

    def start_dst(b, s):
        dstv, s_dst = sets[b][1], sets[b][5]
        pltpu.async_copy(dst_hbm.at[pl.ds(to_cid(s) * _K, _K)], dstv, s_dst)

    def wait_dst(b):
        dstv, s_dst = sets[b][1], sets[b][5]
        pltpu.make_async_copy(dst_hbm.at[pl.ds(0, _K)], dstv, s_dst).wait()

    def start_data(b, s):
        srcv, xg, eav = sets[b][0], sets[b][2], sets[b][3]
        s_g, s_e = sets[b][6], sets[b][7]
        pltpu.async_copy(x_hbm.at[srcv], xg, s_g)
        pltpu.async_copy(eap_hbm.at[pl.ds(to_cid(s) * (_K // _PACK), _K // _PACK)],
                         eav, s_e)

    def wait_data(b):
        srcv, xg, eav = sets[b][0], sets[b][2], sets[b][3]
        s_g, s_e = sets[b][6], sets[b][7]
        pltpu.make_async_copy(x_hbm.at[srcv], xg, s_g).wait()
        pltpu.make_async_copy(eap_hbm.at[pl.ds(0, _K // _PACK)], eav, s_e).wait()

    def compute_scatter(b, s, p):
        dstv, xg, eav, s_m = sets[b][1], sets[b][2], sets[b][3], sets[b][8]

        @pl.loop(0, _K // _PACK)
        def _(pr):
            for a in range(_PACK):
                for cc in range(0, _D, _L):
                    v = (xg[pr * _PACK + a, pl.ds(cc, _L)]
                         + eav[pr, pl.ds(a * _D + cc, _L)])
                    xg[pr * _PACK + a, pl.ds(cc, _L)] = jnp.maximum(v, 0.0)

        pltpu.async_copy(xg, msg_hbm.at[pl.ds(to_cid(s) * _K, _K)], s_m)

        wait_dst(b)

        @pl.when(s >= n_real)
        def _():
            for j in range(_K // _L):
                dstv[pl.ds(j * _L, _L)] = _N + j * _L + lax.iota(jnp.int32, _L)

        for j in range(_K // _L):
            d = dstv[pl.ds(j * _L, _L)] - (p * _RNG)
            ok = (d >= 0) & (d < _RNG)
            pad = _RNG + j * _L + lax.iota(jnp.int32, _L)
            dstv[pl.ds(j * _L, _L)] = jnp.where(ok, d, pad)

        pltpu.sync_copy(xg, agg.at[dstv], add=True)

    def wait_msg(b):
        xg, s_m = sets[b][2], sets[b][8]
        pltpu.make_async_copy(xg, msg_hbm.at[pl.ds(0, _K)], s_m).wait()

    def start_light(b, s, p):
        dstv, xg = sets[b][1], sets[b][2]
        s_dst, s_g = sets[b][5], sets[b][6]
        pltpu.async_copy(dst_hbm.at[pl.ds(to_cid(s) * _K, _K)], dstv, s_dst)
        pltpu.async_copy(msg_hbm.at[pl.ds(to_cid(s) * _K, _K)], xg, s_g)

    def process_light(b, s, p):
        dstv, xg = sets[b][1], sets[b][2]
        s_dst, s_g = sets[b][5], sets[b][6]
        pltpu.make_async_copy(dst_hbm.at[pl.ds(0, _K)], dstv, s_dst).wait()
        pltpu.make_async_copy(msg_hbm.at[pl.ds(0, _K)], xg, s_g).wait()

        @pl.when(s >= n_real)
        def _():
            for j in range(_K // _L):
                dstv[pl.ds(j * _L, _L)] = _N + j * _L + lax.iota(jnp.int32, _L)

        for j in range(_K // _L):
            d = dstv[pl.ds(j * _L, _L)] - (p * _RNG)
            ok = (d >= 0) & (d < _RNG)
            pad = _RNG + j * _L + lax.iota(jnp.int32, _L)
            dstv[pl.ds(j * _L, _L)] = jnp.where(ok, d, pad)

        pltpu.sync_copy(xg, agg.at[dstv], add=True)

    @pl.loop(0, _ZB)
    def _(r):
        for cc in range(0, _D, _L):
            zb[r, pl.ds(cc, _L)] = jnp.zeros((_L,), jnp.float32)

    def copy_out(p):
        @pl.loop(0, _RPS // _ZB)
        def _(j):
            row = sid * _RPS + j * _ZB

            @pl.when(cid == 0)
            def _():
                pltpu.sync_copy(agg.at[pl.ds(row, _ZB)],
                                out0_hbm.at[pl.ds(p * _RNG + row, _ZB)])

            @pl.when(cid == 1)
            def _():
                pltpu.sync_copy(agg.at[pl.ds(row, _ZB)],
                                out1_hbm.at[pl.ds(p * _RNG + row, _ZB)])

    def zero_agg():
        @pl.loop(0, _RPS // _ZB)
        def _(j):
            pltpu.sync_copy(zb, agg.at[pl.ds(sid * _RPS + j * _ZB, _ZB)])

    zero_agg()
    plsc.subcore_barrier()

    start_src(0, 0)
    start_dst(0, 0)
    start_src(1, 1)
    start_dst(1, 1)
    wait_src(0)
    start_data(0, 0)

    @pl.loop(0, (_NSLOT - 1) // 2)
    def _(t):
        c0 = 2 * t
        c1 = c0 + 1
        wait_src(1)

        @pl.when(t > 0)
        def _():
            wait_msg(1)

        start_data(1, c1)
        wait_data(0)
        start_src(0, c0 + 2)
        compute_scatter(0, c0, 0)
        start_dst(0, c0 + 2)
        wait_src(0)
        wait_msg(0)
        start_data(0, c0 + 2)
        wait_data(1)

        @pl.when(c1 + 2 < _NSLOT)
        def _():
            start_src(1, c1 + 2)

        compute_scatter(1, c1, 0)

        @pl.when(c1 + 2 < _NSLOT)
        def _():
            start_dst(1, c1 + 2)

    wait_data(0)
    compute_scatter(0, _NSLOT - 1, 0)
    wait_msg(0)
    wait_msg(1)

    plsc.subcore_barrier()
    copy_out(0)
    plsc.subcore_barrier()

    @pl.loop(1, _NPASS)
    def _(p):
        zero_agg()
        plsc.subcore_barrier()

        start_light(0, 0, p)
        start_light(1, 1, p)

        @pl.loop(0, (_NSLOT - 1) // 2)
        def _(t):
            c0 = 2 * t
            c1 = c0 + 1
            process_light(0, c0, p)
            start_light(0, c0 + 2, p)
            process_light(1, c1, p)

            @pl.when(c1 + 2 < _NSLOT)
            def _():
                start_light(1, c1 + 2, p)

        process_light(0, _NSLOT - 1, p)

        plsc.subcore_barrier()
        copy_out(p)
        plsc.subcore_barrier()


def _sc_aggregate(x, eap, src, dst):
    mesh = plsc.VectorSubcoreMesh(
        core_axis_name="c", subcore_axis_name="s",
        num_cores=_NC, num_subcores=_NS,
    )
    f = pl.kernel(
        _sc_body,
        out_type=[
            jax.ShapeDtypeStruct((_NO, _D), jnp.float32),
            jax.ShapeDtypeStruct((_NO, _D), jnp.float32),
            jax.ShapeDtypeStruct((_E, _D), jnp.float32),
        ],
        mesh=mesh,
        scratch_types=[
            pltpu.VMEM((_K,), jnp.int32),
            pltpu.VMEM((_K,), jnp.int32),
            pltpu.VMEM((_K,), jnp.int32),
            pltpu.VMEM((_K,), jnp.int32),
            pltpu.VMEM((_K, _D), jnp.float32),
            pltpu.VMEM((_K, _D), jnp.float32),
            pltpu.VMEM((_K // _PACK, _PACK * _D), jnp.float32),
            pltpu.VMEM((_K // _PACK, _PACK * _D), jnp.float32),
            pltpu.VMEM((_ZB, _D), jnp.float32),
            pltpu.VMEM_SHARED((_NP, _D), jnp.float32),
            pltpu.SemaphoreType.DMA,
            pltpu.SemaphoreType.DMA,
            pltpu.SemaphoreType.DMA,
            pltpu.SemaphoreType.DMA,
            pltpu.SemaphoreType.DMA,
            pltpu.SemaphoreType.DMA,
            pltpu.SemaphoreType.DMA,
            pltpu.SemaphoreType.DMA,
            pltpu.SemaphoreType.DMA,
            pltpu.SemaphoreType.DMA,
        ],
    )
    return f(x, eap, src, dst)


def _mlp_body(eps_ref, x_ref, p0_ref, p1_ref, w1_ref, c1_ref, w2_ref, c2_ref,
              s3_ref, c3_ref, o_ref):
    a = (1.0 + eps_ref[0]) * x_ref[...] + p0_ref[...] + p1_ref[...]
    h = jnp.dot(a, w1_ref[...], preferred_element_type=jnp.float32) + c1_ref[...]
    h = jnp.maximum(h, 0.0)
    h = jnp.dot(h, w2_ref[...], preferred_element_type=jnp.float32) + c2_ref[...]
    h = jnp.maximum(h, 0.0)
    o_ref[...] = jnp.maximum(h * s3_ref[...] + c3_ref[...], 0.0)


def _mlp(x, p0, p1, eps_p, w1f, c1, w2f, c2, s3, c3):
    bn = 2000
    return pl.pallas_call(
        _mlp_body,
        grid=(_N // bn,),
        in_specs=[
            pl.BlockSpec(memory_space=pltpu.SMEM),
            pl.BlockSpec((bn, _D), lambda i: (i, 0)),
            pl.BlockSpec((bn, _D), lambda i: (i, 0)),
            pl.BlockSpec((bn, _D), lambda i: (i, 0)),
            pl.BlockSpec((_D, _D), lambda i: (0, 0)),
            pl.BlockSpec((1, _D), lambda i: (0, 0)),
            pl.BlockSpec((_D, _D), lambda i: (0, 0)),
            pl.BlockSpec((1, _D), lambda i: (0, 0)),
            pl.BlockSpec((1, _D), lambda i: (0, 0)),
            pl.BlockSpec((1, _D), lambda i: (0, 0)),
        ],
        out_specs=pl.BlockSpec((bn, _D), lambda i: (i, 0)),
        out_shape=jax.ShapeDtypeStruct((_N, _D), jnp.float32),
    )(eps_p.reshape(1), x, p0, p1, w1f, c1, w2f, c2, s3, c3)


def kernel(x, edge_index, edge_attr, W_enc, b_enc, W_lin, b_lin,
           W1, b1, g1, be1, W2, b2, g2, be2, g_bn, be_bn, eps_p):
    Wc = W_lin @ W_enc
    bc = W_lin @ b_enc + b_lin
    eye = jnp.eye(_PACK, dtype=jnp.float32)
    k_big = jnp.einsum("ab,jo->ajbo", eye, Wc.T)
    k_big = k_big.reshape(_PACK * _DE, _PACK * _D).astype(jnp.bfloat16)
    bias_tiled = jnp.tile(bc, _PACK).reshape(1, _PACK * _D)

    inv = 1.0 / jnp.sqrt(1.0 + _BN_EPS)
    w1f = W1.T * (inv * g1)[None, :]
    c1 = ((b1 * inv) * g1 + be1).reshape(1, _D)
    w2f = W2.T * (inv * g2)[None, :]
    c2 = ((b2 * inv) * g2 + be2).reshape(1, _D)
    s3 = (inv * g_bn).reshape(1, _D)
    c3 = be_bn.reshape(1, _D)

    attr_packed = edge_attr.reshape(_EP, _PACK * _DE)
    eap = _edge_encode(attr_packed, k_big, bias_tiled)

    src = edge_index[0]
    dst = edge_index[1]
    p0, p1, _msg = _sc_aggregate(x, eap, src, dst)

    return _mlp(x, p0, p1, eps_p, w1f, c1, w2f, c2, s3, c3)
